# baseline (speedup 1.0000x reference)
import jax
import jax.numpy as jnp
from jax import lax
from jax.experimental import pallas as pl
from jax.experimental.pallas import tpu as pltpu
from jax.experimental.pallas import tpu_sc as plsc

NUM_EXPERTS = 16
BATCH = 128
TOP_K = 2
D_STATE = 16
ROWS = 8192
BKT = BATCH * TOP_K

NC = 2
NS = 16
DT = D_STATE // NC
BK_PER_S = BKT // NS
LANES = 16

SPN = 9
RH = ROWS // 4
RNG = BK_PER_S - SPN
NG = 4 * RNG
NBUF = 3


def _bk(s, j):
    return s * (BK_PER_S // TOP_K) + j // TOP_K, j % TOP_K


def _body(table_hbm, idx_hbm, out_hbm, idx16, shared,
          b0, b1, b2, g0, g1, g2, w0, w1, w2, spsem):
    bufs, gsems, wsems = [b0, b1, b2], [g0, g1, g2], [w0, w1, w2]
    c = lax.axis_index("c")
    s = lax.axis_index("s")

    pltpu.sync_copy(table_hbm.at[s, pl.ds(c * DT, DT)], shared.at[s])

    pltpu.sync_copy(idx_hbm.at[pl.ds(s * BK_PER_S, BK_PER_S)], idx16)
    ids = idx16[...]

    plsc.subcore_barrier()

    spcopies = []
    for j in range(SPN):
        b, k = _bk(s, j)
        spcopies.append(pltpu.async_copy(
            shared.at[ids[j]],
            out_hbm.at[b, k, pl.ds(c * DT, DT)],
            spsem))

    def src(g):
        e = ids[SPN + g // 4]
        return table_hbm.at[e, pl.ds(c * DT, DT), pl.ds((g % 4) * RH, RH)]

    def dst(g):
        b, k = _bk(s, SPN + g // 4)
        return out_hbm.at[b, k, pl.ds(c * DT, DT), pl.ds((g % 4) * RH, RH)]

    gh = [None] * NG
    wh = [None] * NG
    for g in range(NG):
        slot = g % NBUF
        if g >= NBUF:
            wh[g - NBUF].wait()
        gh[g] = pltpu.async_copy(src(g), bufs[slot], gsems[slot])
        if g >= 1:
            gh[g - 1].wait()
            wh[g - 1] = pltpu.async_copy(
                bufs[(g - 1) % NBUF], dst(g - 1), wsems[(g - 1) % NBUF])
    gh[NG - 1].wait()
    wh[NG - 1] = pltpu.async_copy(
        bufs[(NG - 1) % NBUF], dst(NG - 1), wsems[(NG - 1) % NBUF])
    for g in range(max(NG - NBUF, 0), NG):
        wh[g].wait()
    for cp in spcopies:
        cp.wait()


@jax.jit
def _gather_sc(indices_flat, table_t):
    kfn = pl.kernel(
        _body,
        out_type=jax.ShapeDtypeStruct((BATCH, TOP_K, D_STATE, ROWS),
                                      jnp.float32),
        mesh=plsc.VectorSubcoreMesh(
            core_axis_name="c", subcore_axis_name="s",
            num_cores=NC, num_subcores=NS),
        scratch_types=(
            [pltpu.VMEM((LANES,), jnp.int32),
             pltpu.VMEM_SHARED((NUM_EXPERTS, DT, ROWS), jnp.float32)]
            + [pltpu.VMEM((DT, RH), jnp.float32)] * NBUF
            + [pltpu.SemaphoreType.DMA] * (2 * NBUF)
            + [pltpu.SemaphoreType.DMA]
        ),
        compiler_params=pltpu.CompilerParams(needs_layout_passes=False),
    )
    return kfn(table_t, indices_flat)


def kernel(indices, A_experts):
    idx = indices.reshape(BKT).astype(jnp.int32)
    table_t = jnp.transpose(A_experts, (0, 2, 1))
    out_t = _gather_sc(idx, table_t)
    return jnp.transpose(out_t, (0, 1, 3, 2))


# speedup vs baseline: 1.0424x; 1.0424x over previous
import jax
import jax.numpy as jnp
from jax import lax
from jax.experimental import pallas as pl
from jax.experimental.pallas import tpu as pltpu
from jax.experimental.pallas import tpu_sc as plsc

NUM_EXPERTS = 16
BATCH = 128
TOP_K = 2
D_STATE = 16
ROWS = 8192
BKT = BATCH * TOP_K

NC = 2
NS = 16
DT = D_STATE // NC
BK_PER_S = BKT // NS
LANES = 16


def _body(table_hbm, idx_hbm, out_hbm, idx16, shared, sem):
    c = lax.axis_index("c")
    s = lax.axis_index("s")

    pltpu.sync_copy(table_hbm.at[s, pl.ds(c * DT, DT)], shared.at[s])

    pltpu.sync_copy(idx_hbm.at[pl.ds(s * BK_PER_S, BK_PER_S)], idx16)
    ids = idx16[...]

    plsc.subcore_barrier()

    copies = []
    for j in range(BK_PER_S):
        b = s * (BK_PER_S // TOP_K) + j // TOP_K
        k = j % TOP_K
        copies.append(pltpu.async_copy(
            shared.at[ids[j]],
            out_hbm.at[b, k, pl.ds(c * DT, DT)],
            sem))
    for cp in copies:
        cp.wait()


@jax.jit
def _gather_sc(indices_flat, table_t):
    kfn = pl.kernel(
        _body,
        out_type=jax.ShapeDtypeStruct((BATCH, TOP_K, D_STATE, ROWS),
                                      jnp.float32),
        mesh=plsc.VectorSubcoreMesh(
            core_axis_name="c", subcore_axis_name="s",
            num_cores=NC, num_subcores=NS),
        scratch_types=[
            pltpu.VMEM((LANES,), jnp.int32),
            pltpu.VMEM_SHARED((NUM_EXPERTS, DT, ROWS), jnp.float32),
            pltpu.SemaphoreType.DMA,
        ],
        compiler_params=pltpu.CompilerParams(needs_layout_passes=False),
    )
    return kfn(table_t, indices_flat)


def kernel(indices, A_experts):
    idx = indices.reshape(BKT).astype(jnp.int32)
    table_t = jnp.transpose(A_experts, (0, 2, 1))
    out_t = _gather_sc(idx, table_t)
    return jnp.transpose(out_t, (0, 1, 3, 2))
